# Initial kernel scaffold; baseline (speedup 1.0000x reference)
import jax
import jax.numpy as jnp
from jax.experimental import pallas as pl


def kernel(token_ids, weight):
    raise NotImplementedError("write your pallas kernel here")


# speedup vs baseline: 3.3268x; 3.3268x over previous
import functools

import jax
import jax.numpy as jnp
from jax import lax
from jax.experimental import pallas as pl
from jax.experimental.pallas import tpu as pltpu
from jax.experimental.pallas import tpu_sc as plsc

NUM_CORES = 2
NUM_SUBCORES = 16
NUM_WORKERS = NUM_CORES * NUM_SUBCORES
CHUNK = 128


@functools.partial(jax.jit, static_argnames=("n_chunks", "dim"))
def _embedding_gather(weight, idx, n_chunks, dim):
    b_per_w = n_chunks * CHUNK
    total = NUM_WORKERS * b_per_w
    mesh = plsc.VectorSubcoreMesh(core_axis_name="c", subcore_axis_name="s")

    @functools.partial(
        pl.kernel,
        mesh=mesh,
        out_type=jax.ShapeDtypeStruct((total, dim), jnp.float32),
        scratch_types=[
            pltpu.VMEM((n_chunks, CHUNK), jnp.int32),
            pltpu.VMEM((CHUNK, dim), jnp.float32),
            pltpu.VMEM((CHUNK, dim), jnp.float32),
            pltpu.SemaphoreType.DMA,
            pltpu.SemaphoreType.DMA,
        ],
    )
    def k(table_hbm, idx_hbm, out_hbm, idx_v, rows0, rows1, sem0, sem1):
        wid = lax.axis_index("s") * NUM_CORES + lax.axis_index("c")
        base = wid * b_per_w
        pltpu.sync_copy(idx_hbm.at[wid], idx_v)

        def gather_start(chunk, buf, sem):
            pltpu.async_copy(table_hbm.at[idx_v.at[chunk]], buf, sem)

        def gather_wait(chunk, buf, sem):
            pltpu.make_async_copy(table_hbm.at[idx_v.at[chunk]], buf, sem).wait()

        def write_out(chunk, buf):
            pltpu.sync_copy(buf, out_hbm.at[pl.ds(base + chunk * CHUNK, CHUNK)])

        gather_start(0, rows0, sem0)

        def body(g, carry):
            c0 = 2 * g
            gather_start(c0 + 1, rows1, sem1)
            gather_wait(c0, rows0, sem0)
            write_out(c0, rows0)

            @pl.when(g + 1 < n_chunks // 2)
            def _():
                gather_start(c0 + 2, rows0, sem0)

            gather_wait(c0 + 1, rows1, sem1)
            write_out(c0 + 1, rows1)
            return carry

        lax.fori_loop(0, n_chunks // 2, body, 0)

    return k(weight, idx)


def kernel(token_ids, weight):
    b0, b1 = token_ids.shape
    _, dim = weight.shape
    total = b0 * b1
    b_per_w = total // NUM_WORKERS
    n_chunks = b_per_w // CHUNK
    idx = token_ids.reshape(NUM_WORKERS, n_chunks, CHUNK).astype(jnp.int32)
    out = _embedding_gather(weight, idx, n_chunks, dim)
    return out.reshape(b0, b1, dim)


# speedup vs baseline: 3.3414x; 1.0044x over previous
import functools

import jax
import jax.numpy as jnp
from jax import lax
from jax.experimental import pallas as pl
from jax.experimental.pallas import tpu as pltpu
from jax.experimental.pallas import tpu_sc as plsc

NUM_CORES = 2
NUM_SUBCORES = 16
NUM_WORKERS = NUM_CORES * NUM_SUBCORES
CHUNK = 128
NBUF = 5
GLOOK = 3


@functools.partial(jax.jit, static_argnames=("n_chunks", "dim"))
def _embedding_gather(weight, idx, n_chunks, dim):
    b_per_w = n_chunks * CHUNK
    total = NUM_WORKERS * b_per_w
    n_groups = n_chunks // NBUF
    mesh = plsc.VectorSubcoreMesh(core_axis_name="c", subcore_axis_name="s")

    @functools.partial(
        pl.kernel,
        mesh=mesh,
        out_type=jax.ShapeDtypeStruct((total, dim), jnp.float32),
        scratch_types=[
            pltpu.VMEM((n_chunks, CHUNK), jnp.int32),
            pltpu.VMEM((NBUF, CHUNK, dim), jnp.float32),
        ]
        + [pltpu.SemaphoreType.DMA] * (2 * NBUF),
    )
    def k(table_hbm, idx_hbm, out_hbm, idx_v, rows, *sems):
        gsem, wsem = sems[:NBUF], sems[NBUF:]
        wid = lax.axis_index("s") * NUM_CORES + lax.axis_index("c")
        base = wid * b_per_w
        pltpu.sync_copy(idx_hbm.at[wid], idx_v)

        def gather(chunk, b):
            return pltpu.make_async_copy(
                table_hbm.at[idx_v.at[chunk]], rows.at[b], gsem[b]
            )

        def write(chunk, b):
            return pltpu.make_async_copy(
                rows.at[b], out_hbm.at[pl.ds(base + chunk * CHUNK, CHUNK)], wsem[b]
            )

        def step(c, b, wait_w, start_g):
            gather(c, b).wait()
            write(c, b).start()
            if wait_w:
                write(c - 2, (b - 2) % NBUF).wait()
            if start_g:
                gather(c + GLOOK, (b + GLOOK) % NBUF).start()

        for c in range(GLOOK):
            gather(c, c).start()

        for b in range(NBUF):
            step(b, b, b >= 2, True)

        def body(g, carry):
            c0 = g * NBUF
            for b in range(NBUF):
                step(c0 + b, b, True, True)
            return carry

        lax.fori_loop(1, n_groups - 1, body, 0)

        c0 = (n_groups - 1) * NBUF
        for b in range(NBUF):
            step(c0 + b, b, True, c0 + b + GLOOK < n_chunks)

        write(n_chunks - 2, (n_chunks - 2) % NBUF).wait()
        write(n_chunks - 1, (n_chunks - 1) % NBUF).wait()

    return k(weight, idx)


def kernel(token_ids, weight):
    b0, b1 = token_ids.shape
    _, dim = weight.shape
    total = b0 * b1
    b_per_w = total // NUM_WORKERS
    n_chunks = b_per_w // CHUNK
    idx = token_ids.reshape(NUM_WORKERS, n_chunks, CHUNK).astype(jnp.int32)
    out = _embedding_gather(weight, idx, n_chunks, dim)
    return out.reshape(b0, b1, dim)


# speedup vs baseline: 5.9736x; 1.7878x over previous
import functools

import jax
import jax.numpy as jnp
from jax import lax
from jax.experimental import pallas as pl
from jax.experimental.pallas import tpu as pltpu
from jax.experimental.pallas import tpu_sc as plsc

NUM_CORES = 2
NUM_SUBCORES = 16
NUM_WORKERS = NUM_CORES * NUM_SUBCORES
NBUF = 8
GLOOK = 5
WLAG = NBUF - GLOOK


@jax.jit
def _embedding_gather(weight, token_ids):
    batch, seq = token_ids.shape
    _, dim = weight.shape
    n_chunks = batch // NUM_WORKERS
    n_groups = n_chunks // NBUF
    mesh = plsc.VectorSubcoreMesh(core_axis_name="c", subcore_axis_name="s")

    @functools.partial(
        pl.kernel,
        mesh=mesh,
        out_type=jax.ShapeDtypeStruct((batch, seq, dim), jnp.float32),
        scratch_types=[pltpu.VMEM((n_chunks, seq), jnp.int32)]
        + [pltpu.VMEM((seq, dim), jnp.float32)] * NBUF
        + [pltpu.SemaphoreType.DMA] * (2 * NBUF),
        compiler_params=pltpu.CompilerParams(use_tc_tiling_on_sc=True),
    )
    def k(table_hbm, tok_hbm, out_hbm, idx_v, *rows_and_sems):
        rows = rows_and_sems[:NBUF]
        gsem = rows_and_sems[NBUF : 2 * NBUF]
        wsem = rows_and_sems[2 * NBUF :]
        wid = lax.axis_index("s") * NUM_CORES + lax.axis_index("c")
        base = wid * n_chunks
        pltpu.sync_copy(tok_hbm.at[pl.ds(base, n_chunks)], idx_v)

        def gather(chunk, b):
            return pltpu.make_async_copy(
                table_hbm.at[idx_v.at[chunk]], rows[b], gsem[b]
            )

        def write(chunk, b):
            return pltpu.make_async_copy(rows[b], out_hbm.at[base + chunk], wsem[b])

        def step(c, b, wait_w, start_g):
            gather(c, b).wait()
            write(c, b).start()
            if wait_w:
                write(c - WLAG, (b - WLAG) % NBUF).wait()
            if start_g:
                gather(c + GLOOK, (b + GLOOK) % NBUF).start()

        for c in range(GLOOK):
            gather(c, c).start()

        for b in range(NBUF):
            step(b, b, b >= WLAG, True)

        def body(g, carry):
            c0 = g * NBUF
            for b in range(NBUF):
                step(c0 + b, b, True, True)
            return carry

        lax.fori_loop(1, n_groups - 1, body, 0)

        c0 = (n_groups - 1) * NBUF
        for b in range(NBUF):
            step(c0 + b, b, True, b + GLOOK < NBUF)

        for c in range(n_chunks - WLAG, n_chunks):
            write(c, c % NBUF).wait()

    return k(weight, token_ids)


def kernel(token_ids, weight):
    return _embedding_gather(weight, token_ids.astype(jnp.int32))


# speedup vs baseline: 10.7574x; 1.8008x over previous
import functools

import jax
import jax.numpy as jnp
from jax import lax
from jax.experimental import pallas as pl
from jax.experimental.pallas import tpu as pltpu
from jax.experimental.pallas import tpu_sc as plsc

NUM_CORES = 2
NUM_SUBCORES = 16
NUM_WORKERS = NUM_CORES * NUM_SUBCORES
NBUF = 5
GLOOK = 3
WLAG = NBUF - GLOOK


@jax.jit
def _embedding_lookup(weight, token_ids):
    tok_t = token_ids.astype(jnp.int32).T
    seq, batch = tok_t.shape
    _, dim = weight.shape
    n_chunks = seq
    n_groups = n_chunks // NBUF
    blk = batch // NUM_WORKERS
    mesh = plsc.VectorSubcoreMesh(core_axis_name="c", subcore_axis_name="s")

    @functools.partial(
        pl.kernel,
        mesh=mesh,
        out_type=jax.ShapeDtypeStruct((seq, batch, dim), jnp.float32),
        scratch_types=[pltpu.VMEM((seq, blk), jnp.int32)]
        + [pltpu.VMEM((blk, dim), jnp.float32)] * NBUF
        + [pltpu.SemaphoreType.DMA] * (2 * NBUF),
        compiler_params=pltpu.CompilerParams(use_tc_tiling_on_sc=True),
    )
    def k(table_hbm, tok_hbm, out_hbm, idx_v, *rows_and_sems):
        rows = rows_and_sems[:NBUF]
        gsem = rows_and_sems[NBUF : 2 * NBUF]
        wsem = rows_and_sems[2 * NBUF :]
        wid = lax.axis_index("s") * NUM_CORES + lax.axis_index("c")
        b0 = wid * blk
        pltpu.sync_copy(tok_hbm.at[:, pl.ds(b0, blk)], idx_v)

        def gather(chunk, b):
            return pltpu.make_async_copy(
                table_hbm.at[idx_v.at[chunk]], rows[b], gsem[b]
            )

        def write(chunk, b):
            return pltpu.make_async_copy(
                rows[b], out_hbm.at[chunk, pl.ds(b0, blk)], wsem[b]
            )

        def step(c, b, wait_w, start_g):
            gather(c, b).wait()
            write(c, b).start()
            if wait_w:
                write(c - WLAG, (b - WLAG) % NBUF).wait()
            if start_g:
                gather(c + GLOOK, (b + GLOOK) % NBUF).start()

        for c in range(GLOOK):
            gather(c, c).start()

        for b in range(NBUF):
            step(b, b, b >= WLAG, True)

        def body(g, carry):
            c0 = g * NBUF
            for b in range(NBUF):
                step(c0 + b, b, True, True)
            return carry

        lax.fori_loop(1, n_groups - 1, body, 0)

        c0 = (n_groups - 1) * NBUF
        for b in range(NBUF):
            step(c0 + b, b, True, b + GLOOK < NBUF)

        for c in range(n_chunks - WLAG, n_chunks):
            write(c, c % NBUF).wait()

    out_t = k(weight, tok_t)
    return jnp.transpose(out_t, (1, 0, 2))


def kernel(token_ids, weight):
    return _embedding_lookup(weight, token_ids)


# speedup vs baseline: 10.8003x; 1.0040x over previous
import functools

import jax
import jax.numpy as jnp
from jax import lax
from jax.experimental import pallas as pl
from jax.experimental.pallas import tpu as pltpu
from jax.experimental.pallas import tpu_sc as plsc

NUM_CORES = 2
NUM_SUBCORES = 16
NUM_WORKERS = NUM_CORES * NUM_SUBCORES
NBUF = 5
GLOOK = 4
WLAG = NBUF - GLOOK


@jax.jit
def _embedding_lookup(weight, token_ids):
    tok_t = token_ids.astype(jnp.int32).T
    seq, batch = tok_t.shape
    _, dim = weight.shape
    n_chunks = seq
    n_groups = n_chunks // NBUF
    blk = batch // NUM_WORKERS
    mesh = plsc.VectorSubcoreMesh(core_axis_name="c", subcore_axis_name="s")

    @functools.partial(
        pl.kernel,
        mesh=mesh,
        out_type=jax.ShapeDtypeStruct((seq, batch, dim), jnp.float32),
        scratch_types=[pltpu.VMEM((seq, blk), jnp.int32)]
        + [pltpu.VMEM((blk, dim), jnp.float32)] * NBUF
        + [pltpu.SemaphoreType.DMA] * (2 * NBUF),
        compiler_params=pltpu.CompilerParams(use_tc_tiling_on_sc=True),
    )
    def k(table_hbm, tok_hbm, out_hbm, idx_v, *rows_and_sems):
        rows = rows_and_sems[:NBUF]
        gsem = rows_and_sems[NBUF : 2 * NBUF]
        wsem = rows_and_sems[2 * NBUF :]
        wid = lax.axis_index("s") * NUM_CORES + lax.axis_index("c")
        b0 = wid * blk
        pltpu.sync_copy(tok_hbm.at[:, pl.ds(b0, blk)], idx_v)

        def gather(chunk, b):
            return pltpu.make_async_copy(
                table_hbm.at[idx_v.at[chunk]], rows[b], gsem[b]
            )

        def write(chunk, b):
            return pltpu.make_async_copy(
                rows[b], out_hbm.at[chunk, pl.ds(b0, blk)], wsem[b]
            )

        def step(c, b, wait_w, start_g):
            gather(c, b).wait()
            write(c, b).start()
            if wait_w:
                write(c - WLAG, (b - WLAG) % NBUF).wait()
            if start_g:
                gather(c + GLOOK, (b + GLOOK) % NBUF).start()

        for c in range(GLOOK):
            gather(c, c).start()

        for b in range(NBUF):
            step(b, b, b >= WLAG, True)

        def body(g, carry):
            c0 = g * NBUF
            for b in range(NBUF):
                step(c0 + b, b, True, True)
            return carry

        lax.fori_loop(1, n_groups - 1, body, 0)

        c0 = (n_groups - 1) * NBUF
        for b in range(NBUF):
            step(c0 + b, b, True, b + GLOOK < NBUF)

        for c in range(n_chunks - WLAG, n_chunks):
            write(c, c % NBUF).wait()

    out_t = k(weight, tok_t)
    return jnp.transpose(out_t, (1, 0, 2))


def kernel(token_ids, weight):
    return _embedding_lookup(weight, token_ids)
